# Initial kernel scaffold; baseline (speedup 1.0000x reference)
import jax
import jax.numpy as jnp
from jax.experimental import pallas as pl


def kernel(x, mem_kv, W_q, W_kv, W_out, b_out, null_k, null_v, gate):
    raise NotImplementedError("write your pallas kernel here")


# speedup vs baseline: 16.6624x; 16.6624x over previous
import jax
import jax.numpy as jnp
from jax.experimental import pallas as pl
from jax.experimental.pallas import tpu as pltpu

B, N, DIM = 1, 2048, 1024
HEADS, DIM_HEAD = 16, 64
INNER = HEADS * DIM_HEAD
M_MEM = 4096
K_RETR = 32
SCALE = DIM_HEAD ** -0.5

BLK = 256
NI = N // BLK
NEG = -3.4028235e38


def _bdot(a, b):
    return jnp.dot(a.astype(jnp.bfloat16), b.astype(jnp.bfloat16),
                   preferred_element_type=jnp.float32)


def _kv_kernel(x_ref, wkv_ref, kv_ref):
    kv_ref[...] = _bdot(x_ref[...], wkv_ref[...])


def _attn_kernel(x_ref, wq_ref, kv_ref, mk_ref, mv_ref, nk_ref, nv_ref,
                 gate_ref, wout_ref, bout_ref, out_ref):
    i = pl.program_id(0)
    h = pl.program_id(1)
    q = _bdot(x_ref[...], wq_ref[0])
    k = kv_ref[:, :DIM_HEAD]
    v = kv_ref[:, DIM_HEAD:]

    sim = _bdot(q, k.T) * SCALE
    row = i * BLK + jax.lax.broadcasted_iota(jnp.int32, (BLK, N), 0)
    col = jax.lax.broadcasted_iota(jnp.int32, (BLK, N), 1)
    sim = jnp.where(col <= row, sim, NEG)
    ml = jnp.max(sim, axis=1, keepdims=True)
    p = jnp.exp(sim - ml)
    attn = p / jnp.sum(p, axis=1, keepdims=True)
    loc = _bdot(attn, v)

    qnorm = jnp.clip(jnp.sqrt(jnp.sum(q * q, axis=1, keepdims=True)),
                     1e-12, None)
    s_sel = _bdot(q / qnorm, mk_ref[...].T)
    ki = jax.lax.bitcast_convert_type(s_sel, jnp.int32)
    key = jnp.where(ki < 0, (~ki) ^ jnp.int32(-2147483648), ki)
    t = jnp.full((BLK, 1), jnp.int32(-2147483648))
    for b in range(31, -1, -1):
        step = jnp.int32(-2147483648) if b == 31 else jnp.int32(1 << b)
        cand = t + step
        cnt = jnp.sum((key >= cand).astype(jnp.float32), axis=1, keepdims=True)
        t = jnp.where(cnt >= K_RETR, cand, t)
    sel = key >= t

    s = s_sel * (qnorm * SCALE)
    null_s = _bdot(q, nk_ref[...].T) * SCALE
    m = jnp.maximum(jnp.max(jnp.where(sel, s, NEG), axis=1, keepdims=True),
                    null_s)
    w = jnp.where(sel, jnp.exp(s - m), 0.0)
    wn = jnp.exp(null_s - m)
    denom_m = jnp.sum(w, axis=1, keepdims=True) + wn
    w = w / denom_m
    mem = _bdot(w, mv_ref[...]) + (wn / denom_m) * nv_ref[...]

    g = jax.nn.sigmoid(gate_ref[0, 0, 0])
    part = loc * g + mem * (1.0 - g)
    contrib = _bdot(part, wout_ref[0])

    @pl.when(h == 0)
    def _init():
        out_ref[...] = contrib + bout_ref[...]

    @pl.when(h != 0)
    def _acc():
        out_ref[...] += contrib


@jax.jit
def kernel(x, mem_kv, W_q, W_kv, W_out, b_out, null_k, null_v, gate):
    x2 = x[0]
    mem_k = mem_kv[0, :, 0, :]
    mem_v = mem_kv[0, :, 1, :]
    wq_r = W_q.reshape(DIM, HEADS, DIM_HEAD).transpose(1, 0, 2)
    wout_r = W_out.reshape(HEADS, DIM_HEAD, DIM)

    kv = pl.pallas_call(
        _kv_kernel,
        grid=(1,),
        in_specs=[
            pl.BlockSpec((N, DIM), lambda i: (0, 0)),
            pl.BlockSpec((DIM, 2 * DIM_HEAD), lambda i: (0, 0)),
        ],
        out_specs=pl.BlockSpec((N, 2 * DIM_HEAD), lambda i: (0, 0)),
        out_shape=jax.ShapeDtypeStruct((N, 2 * DIM_HEAD), jnp.float32),
    )(x2, W_kv)

    out = pl.pallas_call(
        _attn_kernel,
        grid=(NI, HEADS),
        in_specs=[
            pl.BlockSpec((BLK, DIM), lambda i, h: (i, 0)),
            pl.BlockSpec((1, DIM, DIM_HEAD), lambda i, h: (h, 0, 0)),
            pl.BlockSpec((N, 2 * DIM_HEAD), lambda i, h: (0, 0)),
            pl.BlockSpec((M_MEM, DIM_HEAD), lambda i, h: (0, 0)),
            pl.BlockSpec((M_MEM, DIM_HEAD), lambda i, h: (0, 0)),
            pl.BlockSpec((1, DIM_HEAD), lambda i, h: (0, 0)),
            pl.BlockSpec((1, DIM_HEAD), lambda i, h: (0, 0)),
            pl.BlockSpec((1, 1, 1), lambda i, h: (h, 0, 0)),
            pl.BlockSpec((1, DIM_HEAD, DIM), lambda i, h: (h, 0, 0)),
            pl.BlockSpec((1, DIM), lambda i, h: (0, 0)),
        ],
        out_specs=pl.BlockSpec((BLK, DIM), lambda i, h: (i, 0)),
        out_shape=jax.ShapeDtypeStruct((N, DIM), jnp.float32),
        compiler_params=pltpu.CompilerParams(
            dimension_semantics=("parallel", "arbitrary"),
        ),
    )(x2, wq_r, kv, mem_k, mem_v,
      null_k.reshape(1, DIM_HEAD), null_v.reshape(1, DIM_HEAD),
      gate, wout_r, b_out.reshape(1, DIM))

    return out[None]


# speedup vs baseline: 19.0302x; 1.1421x over previous
import jax
import jax.numpy as jnp
from jax.experimental import pallas as pl
from jax.experimental.pallas import tpu as pltpu

B, N, DIM = 1, 2048, 1024
HEADS, DIM_HEAD = 16, 64
INNER = HEADS * DIM_HEAD
M_MEM = 4096
K_RETR = 32
SCALE = DIM_HEAD ** -0.5

BLK = 256
NI = N // BLK
NEG = -3.4028235e38


def _bdot(a, b):
    return jnp.dot(a.astype(jnp.bfloat16), b.astype(jnp.bfloat16),
                   preferred_element_type=jnp.float32)


def _kv_kernel(x_ref, wkv_ref, kv_ref):
    kv_ref[...] = _bdot(x_ref[...], wkv_ref[...])


def _attn_kernel(x_ref, wq_ref, kv_ref, mk_ref, mv_ref, nk_ref, nv_ref,
                 gate_ref, wout_ref, bout_ref, out_ref):
    i = pl.program_id(0)
    h = pl.program_id(1)
    q = _bdot(x_ref[...], wq_ref[0])
    k = kv_ref[:, :DIM_HEAD]
    v = kv_ref[:, DIM_HEAD:]

    sim = _bdot(q, k.T) * SCALE
    row = i * BLK + jax.lax.broadcasted_iota(jnp.int32, (BLK, N), 0)
    col = jax.lax.broadcasted_iota(jnp.int32, (BLK, N), 1)
    sim = jnp.where(col <= row, sim, NEG)
    ml = jnp.max(sim, axis=1, keepdims=True)
    p = jnp.exp(sim - ml)
    attn = p / jnp.sum(p, axis=1, keepdims=True)
    loc = _bdot(attn, v)

    qnorm = jnp.clip(jnp.sqrt(jnp.sum(q * q, axis=1, keepdims=True)),
                     1e-12, None)
    s_sel = _bdot(q / qnorm, mk_ref[...].T)
    cmax = jnp.max(s_sel.reshape(BLK, 32, M_MEM // 32), axis=2)
    lo0 = jnp.min(cmax, axis=1, keepdims=True)
    hi0 = jnp.max(cmax, axis=1, keepdims=True)
    kf = float(K_RETR)
    cnt0 = jnp.sum((s_sel >= lo0).astype(jnp.float32), axis=1, keepdims=True)

    def _cond(st):
        it, lo, hi, cnt = st
        return jnp.logical_and(it < 36, jnp.any(cnt != kf))

    def _body(st):
        it, lo, hi, cnt = st
        done = cnt == kf
        mid = lo + 0.5 * (hi - lo)
        c = jnp.sum((s_sel >= mid).astype(jnp.float32), axis=1, keepdims=True)
        ge = c >= kf
        lo2 = jnp.where(done, lo, jnp.where(ge, mid, lo))
        hi2 = jnp.where(done, hi, jnp.where(ge, hi, mid))
        cnt2 = jnp.where(done, cnt, jnp.where(ge, c, cnt))
        return it + 1, lo2, hi2, cnt2

    _, t, _, _ = jax.lax.while_loop(_cond, _body, (0, lo0, hi0, cnt0))
    sel = s_sel >= t

    s = s_sel * (qnorm * SCALE)
    null_s = _bdot(q, nk_ref[...].T) * SCALE
    m = jnp.maximum(jnp.max(jnp.where(sel, s, NEG), axis=1, keepdims=True),
                    null_s)
    w = jnp.where(sel, jnp.exp(s - m), 0.0)
    wn = jnp.exp(null_s - m)
    denom_m = jnp.sum(w, axis=1, keepdims=True) + wn
    w = w / denom_m
    mem = _bdot(w, mv_ref[...]) + (wn / denom_m) * nv_ref[...]

    g = jax.nn.sigmoid(gate_ref[0, 0, 0])
    part = loc * g + mem * (1.0 - g)
    contrib = _bdot(part, wout_ref[0])

    @pl.when(h == 0)
    def _init():
        out_ref[...] = contrib + bout_ref[...]

    @pl.when(h != 0)
    def _acc():
        out_ref[...] += contrib


@jax.jit
def kernel(x, mem_kv, W_q, W_kv, W_out, b_out, null_k, null_v, gate):
    x2 = x[0]
    mem_k = mem_kv[0, :, 0, :]
    mem_v = mem_kv[0, :, 1, :]
    wq_r = W_q.reshape(DIM, HEADS, DIM_HEAD).transpose(1, 0, 2)
    wout_r = W_out.reshape(HEADS, DIM_HEAD, DIM)

    kv = pl.pallas_call(
        _kv_kernel,
        grid=(1,),
        in_specs=[
            pl.BlockSpec((N, DIM), lambda i: (0, 0)),
            pl.BlockSpec((DIM, 2 * DIM_HEAD), lambda i: (0, 0)),
        ],
        out_specs=pl.BlockSpec((N, 2 * DIM_HEAD), lambda i: (0, 0)),
        out_shape=jax.ShapeDtypeStruct((N, 2 * DIM_HEAD), jnp.float32),
    )(x2, W_kv)

    out = pl.pallas_call(
        _attn_kernel,
        grid=(NI, HEADS),
        in_specs=[
            pl.BlockSpec((BLK, DIM), lambda i, h: (i, 0)),
            pl.BlockSpec((1, DIM, DIM_HEAD), lambda i, h: (h, 0, 0)),
            pl.BlockSpec((N, 2 * DIM_HEAD), lambda i, h: (0, 0)),
            pl.BlockSpec((M_MEM, DIM_HEAD), lambda i, h: (0, 0)),
            pl.BlockSpec((M_MEM, DIM_HEAD), lambda i, h: (0, 0)),
            pl.BlockSpec((1, DIM_HEAD), lambda i, h: (0, 0)),
            pl.BlockSpec((1, DIM_HEAD), lambda i, h: (0, 0)),
            pl.BlockSpec((1, 1, 1), lambda i, h: (h, 0, 0)),
            pl.BlockSpec((1, DIM_HEAD, DIM), lambda i, h: (h, 0, 0)),
            pl.BlockSpec((1, DIM), lambda i, h: (0, 0)),
        ],
        out_specs=pl.BlockSpec((BLK, DIM), lambda i, h: (i, 0)),
        out_shape=jax.ShapeDtypeStruct((N, DIM), jnp.float32),
        compiler_params=pltpu.CompilerParams(
            dimension_semantics=("parallel", "arbitrary"),
        ),
    )(x2, wq_r, kv, mem_k, mem_v,
      null_k.reshape(1, DIM_HEAD), null_v.reshape(1, DIM_HEAD),
      gate, wout_r, b_out.reshape(1, DIM))

    return out[None]


# speedup vs baseline: 22.8916x; 1.2029x over previous
import jax
import jax.numpy as jnp
from jax.experimental import pallas as pl
from jax.experimental.pallas import tpu as pltpu

B, N, DIM = 1, 2048, 1024
HEADS, DIM_HEAD = 16, 64
INNER = HEADS * DIM_HEAD
M_MEM = 4096
K_RETR = 32
SCALE = DIM_HEAD ** -0.5

BLK = 256
NI = N // BLK
NEG = -3.4028235e38


def _bdot(a, b):
    return jnp.dot(a.astype(jnp.bfloat16), b.astype(jnp.bfloat16),
                   preferred_element_type=jnp.float32)


def _kv_kernel(x_ref, wkv_ref, kv_ref):
    kv_ref[...] = _bdot(x_ref[...], wkv_ref[...])


def _attn_kernel(x_ref, wq_ref, kv_ref, mk_ref, mv_ref, nk_ref, nv_ref,
                 gate_ref, wout_ref, bout_ref, out_ref):
    i = pl.program_id(0)
    h = pl.program_id(1)
    q = _bdot(x_ref[...], wq_ref[0])
    k = kv_ref[:, :DIM_HEAD]
    v = kv_ref[:, DIM_HEAD:]

    sim = _bdot(q, k.T) * SCALE
    row = i * BLK + jax.lax.broadcasted_iota(jnp.int32, (BLK, N), 0)
    col = jax.lax.broadcasted_iota(jnp.int32, (BLK, N), 1)
    sim = jnp.where(col <= row, sim, NEG)
    ml = jnp.max(sim, axis=1, keepdims=True)
    p = jnp.exp(sim - ml)
    attn = p / jnp.sum(p, axis=1, keepdims=True)
    loc = _bdot(attn, v)

    qnorm = jnp.clip(jnp.sqrt(jnp.sum(q * q, axis=1, keepdims=True)),
                     1e-12, None)
    s_sel = _bdot(q / qnorm, mk_ref[...].T)
    cmax = jnp.max(s_sel.reshape(BLK, 32, M_MEM // 32), axis=2)
    lo0 = jnp.min(cmax, axis=1, keepdims=True)
    hi0 = jnp.max(cmax, axis=1, keepdims=True)
    kf = float(K_RETR)
    cnt0 = jnp.sum((s_sel >= lo0).astype(jnp.float32), axis=1, keepdims=True)
    chi0 = jnp.ones_like(cnt0)

    def _step(lo, hi, cnt, chi, mid):
        done = cnt == kf
        c = jnp.sum((s_sel >= mid).astype(jnp.float32), axis=1, keepdims=True)
        ge = c >= kf
        lo2 = jnp.where(done, lo, jnp.where(ge, mid, lo))
        hi2 = jnp.where(done, hi, jnp.where(ge, hi, mid))
        cnt2 = jnp.where(done, cnt, jnp.where(ge, c, cnt))
        chi2 = jnp.where(done, chi, jnp.where(ge, chi, c))
        return lo2, hi2, cnt2, chi2

    def _interp_mid(lo, hi, cnt, chi):
        num = jnp.log(jnp.maximum(cnt, kf + 1.0) / kf)
        den = jnp.maximum(jnp.log(jnp.maximum(cnt, kf + 1.0)
                                  / jnp.maximum(chi, 1.0)), 1e-6)
        frac = jnp.clip(num / den, 0.12, 0.88)
        return lo + frac * (hi - lo)

    lo, hi, cnt, chi = lo0, hi0, cnt0, chi0
    for _ in range(10):
        lo, hi, cnt, chi = _step(lo, hi, cnt, chi, _interp_mid(lo, hi, cnt, chi))

    def _cond(st):
        it, lo, hi, cnt, chi = st
        return jnp.logical_and(it < 30, jnp.any(cnt != kf))

    def _body(st):
        it, lo, hi, cnt, chi = st
        lo, hi, cnt, chi = _step(lo, hi, cnt, chi, lo + 0.5 * (hi - lo))
        return it + 1, lo, hi, cnt, chi

    _, t, _, _, _ = jax.lax.while_loop(_cond, _body, (0, lo, hi, cnt, chi))
    sel = s_sel >= t

    z = jnp.where(sel, s_sel * (qnorm * SCALE), NEG)
    null_s = _bdot(q, nk_ref[...].T) * SCALE
    m = jnp.maximum(jnp.max(z, axis=1, keepdims=True), null_s)
    w = jnp.exp(z - m)
    wn = jnp.exp(null_s - m)
    denom_m = jnp.sum(w, axis=1, keepdims=True) + wn
    w = w / denom_m
    mem = _bdot(w, mv_ref[...]) + (wn / denom_m) * nv_ref[...]

    g = jax.nn.sigmoid(gate_ref[0, 0, 0])
    part = loc * g + mem * (1.0 - g)
    contrib = _bdot(part, wout_ref[0])

    @pl.when(h == 0)
    def _init():
        out_ref[...] = contrib + bout_ref[...]

    @pl.when(h != 0)
    def _acc():
        out_ref[...] += contrib


@jax.jit
def kernel(x, mem_kv, W_q, W_kv, W_out, b_out, null_k, null_v, gate):
    x2 = x[0]
    mem_k = mem_kv[0, :, 0, :]
    mem_v = mem_kv[0, :, 1, :]
    wq_r = W_q.reshape(DIM, HEADS, DIM_HEAD).transpose(1, 0, 2)
    wout_r = W_out.reshape(HEADS, DIM_HEAD, DIM)

    kv = pl.pallas_call(
        _kv_kernel,
        grid=(1,),
        in_specs=[
            pl.BlockSpec((N, DIM), lambda i: (0, 0)),
            pl.BlockSpec((DIM, 2 * DIM_HEAD), lambda i: (0, 0)),
        ],
        out_specs=pl.BlockSpec((N, 2 * DIM_HEAD), lambda i: (0, 0)),
        out_shape=jax.ShapeDtypeStruct((N, 2 * DIM_HEAD), jnp.float32),
    )(x2, W_kv)

    out = pl.pallas_call(
        _attn_kernel,
        grid=(NI, HEADS),
        in_specs=[
            pl.BlockSpec((BLK, DIM), lambda i, h: (i, 0)),
            pl.BlockSpec((1, DIM, DIM_HEAD), lambda i, h: (h, 0, 0)),
            pl.BlockSpec((N, 2 * DIM_HEAD), lambda i, h: (0, 0)),
            pl.BlockSpec((M_MEM, DIM_HEAD), lambda i, h: (0, 0)),
            pl.BlockSpec((M_MEM, DIM_HEAD), lambda i, h: (0, 0)),
            pl.BlockSpec((1, DIM_HEAD), lambda i, h: (0, 0)),
            pl.BlockSpec((1, DIM_HEAD), lambda i, h: (0, 0)),
            pl.BlockSpec((1, 1, 1), lambda i, h: (h, 0, 0)),
            pl.BlockSpec((1, DIM_HEAD, DIM), lambda i, h: (h, 0, 0)),
            pl.BlockSpec((1, DIM), lambda i, h: (0, 0)),
        ],
        out_specs=pl.BlockSpec((BLK, DIM), lambda i, h: (i, 0)),
        out_shape=jax.ShapeDtypeStruct((N, DIM), jnp.float32),
        compiler_params=pltpu.CompilerParams(
            dimension_semantics=("parallel", "arbitrary"),
        ),
    )(x2, wq_r, kv, mem_k, mem_v,
      null_k.reshape(1, DIM_HEAD), null_v.reshape(1, DIM_HEAD),
      gate, wout_r, b_out.reshape(1, DIM))

    return out[None]


# speedup vs baseline: 25.7074x; 1.1230x over previous
import jax
import jax.numpy as jnp
from jax.experimental import pallas as pl
from jax.experimental.pallas import tpu as pltpu

B, N, DIM = 1, 2048, 1024
HEADS, DIM_HEAD = 16, 64
INNER = HEADS * DIM_HEAD
M_MEM = 4096
K_RETR = 32
SCALE = DIM_HEAD ** -0.5

BLK = 256
NI = N // BLK
NEG = -3.4028235e38


def _bdot(a, b):
    return jnp.dot(a.astype(jnp.bfloat16), b.astype(jnp.bfloat16),
                   preferred_element_type=jnp.float32)


def _kv_kernel(x_ref, wkv_ref, kv_ref):
    kv_ref[...] = _bdot(x_ref[...], wkv_ref[...])


def _attn_kernel(x_ref, wq_ref, kv_ref, mk_ref, mv_ref, nk_ref, nv_ref,
                 gate_ref, wout_ref, bout_ref, out_ref):
    i = pl.program_id(0)
    h = pl.program_id(1)
    q = _bdot(x_ref[...], wq_ref[0])
    k = kv_ref[:, :DIM_HEAD]
    v = kv_ref[:, DIM_HEAD:]

    sim = _bdot(q, k.T) * SCALE
    row = i * BLK + jax.lax.broadcasted_iota(jnp.int32, (BLK, N), 0)
    col = jax.lax.broadcasted_iota(jnp.int32, (BLK, N), 1)
    sim = jnp.where(col <= row, sim, NEG)
    ml = jnp.max(sim, axis=1, keepdims=True)
    p = jnp.exp(sim - ml)
    attn = p / jnp.sum(p, axis=1, keepdims=True)
    loc = _bdot(attn, v)

    qnorm = jnp.clip(jnp.sqrt(jnp.sum(q * q, axis=1, keepdims=True)),
                     1e-12, None)
    s_sel = _bdot(q / qnorm, mk_ref[...].T)
    cmax = jnp.max(s_sel.reshape(BLK, 32, M_MEM // 32), axis=2)
    lo0 = jnp.min(cmax, axis=1, keepdims=True)
    hi0 = jnp.max(cmax, axis=1, keepdims=True)
    kf = float(K_RETR)
    cnt0 = jnp.sum((s_sel >= lo0).astype(jnp.float32), axis=1, keepdims=True)
    chi0 = jnp.ones_like(cnt0)

    kf2 = kf + 2.0

    def _step(lo, hi, cnt, chi):
        done = cnt <= kf2
        num = jnp.log(jnp.maximum(cnt, kf + 1.0) / kf)
        den = jnp.maximum(jnp.log(jnp.maximum(cnt, kf + 1.0)
                                  / jnp.maximum(chi, 1.0)), 1e-6)
        frac = jnp.clip(num / den, 0.12, 0.88)
        mid = lo + frac * (hi - lo)
        c = jnp.sum((s_sel >= mid).astype(jnp.float32), axis=1, keepdims=True)
        ge = c >= kf
        lo2 = jnp.where(done, lo, jnp.where(ge, mid, lo))
        hi2 = jnp.where(done, hi, jnp.where(ge, hi, mid))
        cnt2 = jnp.where(done, cnt, jnp.where(ge, c, cnt))
        chi2 = jnp.where(done, chi, jnp.where(ge, chi, c))
        return lo2, hi2, cnt2, chi2

    lo, hi, cnt, chi = lo0, hi0, cnt0, chi0
    for _ in range(8):
        lo, hi, cnt, chi = _step(lo, hi, cnt, chi)

    def _cond(st):
        it, lo, hi, cnt, chi = st
        return jnp.logical_and(it < 30, jnp.any(cnt > kf2))

    def _body(st):
        it, lo, hi, cnt, chi = st
        lo, hi, cnt, chi = _step(lo, hi, cnt, chi)
        return it + 1, lo, hi, cnt, chi

    _, lo, _, cnt, _ = jax.lax.while_loop(_cond, _body, (0, lo, hi, cnt, chi))
    INF = 3.4028235e38
    u1 = jnp.min(jnp.where(s_sel >= lo, s_sel, INF), axis=1, keepdims=True)
    u2 = jnp.min(jnp.where(s_sel > u1, s_sel, INF), axis=1, keepdims=True)
    u3 = jnp.min(jnp.where(s_sel > u2, s_sel, INF), axis=1, keepdims=True)
    t = jnp.where(cnt == kf, lo, jnp.where(cnt == kf + 1.0, u2, u3))
    t = jnp.where(t >= INF, lo, t)
    z = jnp.where(s_sel >= t, s_sel * (qnorm * SCALE), NEG)
    null_s = _bdot(q, nk_ref[...].T) * SCALE
    m = jnp.maximum(jnp.max(z, axis=1, keepdims=True), null_s)
    w = jnp.exp(z - m)
    wn = jnp.exp(null_s - m)
    denom_m = jnp.sum(w, axis=1, keepdims=True) + wn
    w = w / denom_m
    mem = _bdot(w, mv_ref[...]) + (wn / denom_m) * nv_ref[...]

    g = jax.nn.sigmoid(gate_ref[0, 0, 0])
    part = loc * g + mem * (1.0 - g)
    contrib = _bdot(part, wout_ref[0])

    @pl.when(h == 0)
    def _init():
        out_ref[...] = contrib + bout_ref[...]

    @pl.when(h != 0)
    def _acc():
        out_ref[...] += contrib


@jax.jit
def kernel(x, mem_kv, W_q, W_kv, W_out, b_out, null_k, null_v, gate):
    x2 = x[0]
    mem_k = mem_kv[0, :, 0, :]
    mem_v = mem_kv[0, :, 1, :]
    wq_r = W_q.reshape(DIM, HEADS, DIM_HEAD).transpose(1, 0, 2)
    wout_r = W_out.reshape(HEADS, DIM_HEAD, DIM)

    kv = pl.pallas_call(
        _kv_kernel,
        grid=(1,),
        in_specs=[
            pl.BlockSpec((N, DIM), lambda i: (0, 0)),
            pl.BlockSpec((DIM, 2 * DIM_HEAD), lambda i: (0, 0)),
        ],
        out_specs=pl.BlockSpec((N, 2 * DIM_HEAD), lambda i: (0, 0)),
        out_shape=jax.ShapeDtypeStruct((N, 2 * DIM_HEAD), jnp.float32),
    )(x2, W_kv)

    out = pl.pallas_call(
        _attn_kernel,
        grid=(NI, HEADS),
        in_specs=[
            pl.BlockSpec((BLK, DIM), lambda i, h: (i, 0)),
            pl.BlockSpec((1, DIM, DIM_HEAD), lambda i, h: (h, 0, 0)),
            pl.BlockSpec((N, 2 * DIM_HEAD), lambda i, h: (0, 0)),
            pl.BlockSpec((M_MEM, DIM_HEAD), lambda i, h: (0, 0)),
            pl.BlockSpec((M_MEM, DIM_HEAD), lambda i, h: (0, 0)),
            pl.BlockSpec((1, DIM_HEAD), lambda i, h: (0, 0)),
            pl.BlockSpec((1, DIM_HEAD), lambda i, h: (0, 0)),
            pl.BlockSpec((1, 1, 1), lambda i, h: (h, 0, 0)),
            pl.BlockSpec((1, DIM_HEAD, DIM), lambda i, h: (h, 0, 0)),
            pl.BlockSpec((1, DIM), lambda i, h: (0, 0)),
        ],
        out_specs=pl.BlockSpec((BLK, DIM), lambda i, h: (i, 0)),
        out_shape=jax.ShapeDtypeStruct((N, DIM), jnp.float32),
        compiler_params=pltpu.CompilerParams(
            dimension_semantics=("parallel", "arbitrary"),
        ),
    )(x2, wq_r, kv, mem_k, mem_v,
      null_k.reshape(1, DIM_HEAD), null_v.reshape(1, DIM_HEAD),
      gate, wout_r, b_out.reshape(1, DIM))

    return out[None]


# speedup vs baseline: 25.7845x; 1.0030x over previous
import jax
import jax.numpy as jnp
from jax.experimental import pallas as pl
from jax.experimental.pallas import tpu as pltpu

B, N, DIM = 1, 2048, 1024
HEADS, DIM_HEAD = 16, 64
INNER = HEADS * DIM_HEAD
M_MEM = 4096
K_RETR = 32
SCALE = DIM_HEAD ** -0.5

BLK = 256
NI = N // BLK
NEG = -3.4028235e38


def _bdot(a, b):
    return jnp.dot(a.astype(jnp.bfloat16), b.astype(jnp.bfloat16),
                   preferred_element_type=jnp.float32)


def _kv_kernel(x_ref, wkv_ref, kv_ref):
    kv_ref[...] = _bdot(x_ref[...], wkv_ref[...])


def _attn_kernel(x_ref, wq_ref, kv_ref, mk_ref, mv_ref, nk_ref, nv_ref,
                 gate_ref, wout_ref, bout_ref, out_ref):
    i = pl.program_id(0)
    h = pl.program_id(1)
    q = _bdot(x_ref[...], wq_ref[0])
    k = kv_ref[:, :DIM_HEAD]
    v = kv_ref[:, DIM_HEAD:]

    sim = _bdot(q, k.T) * SCALE
    row = i * BLK + jax.lax.broadcasted_iota(jnp.int32, (BLK, N), 0)
    col = jax.lax.broadcasted_iota(jnp.int32, (BLK, N), 1)
    sim = jnp.where(col <= row, sim, NEG)
    ml = jnp.max(sim, axis=1, keepdims=True)
    p = jnp.exp(sim - ml)
    attn = p / jnp.sum(p, axis=1, keepdims=True)
    loc = _bdot(attn, v)

    qnorm = jnp.clip(jnp.sqrt(jnp.sum(q * q, axis=1, keepdims=True)),
                     1e-12, None)
    s_sel = _bdot(q / qnorm, mk_ref[...].T)
    cmax = jnp.max(s_sel.reshape(BLK, 32, M_MEM // 32), axis=2)
    lo0 = jnp.min(cmax, axis=1, keepdims=True)
    hi0 = jnp.max(cmax, axis=1, keepdims=True)
    kf = float(K_RETR)
    cnt0 = jnp.full_like(lo0, 48.0)
    chi0 = jnp.ones_like(cnt0)

    kf2 = kf + 2.0

    def _step(lo, hi, cnt, chi):
        done = cnt <= kf2
        num = jnp.log(jnp.maximum(cnt, kf + 1.0) / kf)
        den = jnp.maximum(jnp.log(jnp.maximum(cnt, kf + 1.0)
                                  / jnp.maximum(chi, 1.0)), 1e-6)
        frac = jnp.clip(num / den, 0.12, 0.88)
        mid = lo + frac * (hi - lo)
        c = jnp.sum((s_sel >= mid).astype(jnp.float32), axis=1, keepdims=True)
        ge = c >= kf
        lo2 = jnp.where(done, lo, jnp.where(ge, mid, lo))
        hi2 = jnp.where(done, hi, jnp.where(ge, hi, mid))
        cnt2 = jnp.where(done, cnt, jnp.where(ge, c, cnt))
        chi2 = jnp.where(done, chi, jnp.where(ge, chi, c))
        return lo2, hi2, cnt2, chi2

    lo, hi, cnt, chi = lo0, hi0, cnt0, chi0
    for _ in range(8):
        lo, hi, cnt, chi = _step(lo, hi, cnt, chi)

    def _cond(st):
        it, lo, hi, cnt, chi = st
        return jnp.logical_and(it < 30, jnp.any(cnt > kf2))

    def _body(st):
        it, lo, hi, cnt, chi = st
        lo, hi, cnt, chi = _step(lo, hi, cnt, chi)
        return it + 1, lo, hi, cnt, chi

    _, lo, _, cnt, _ = jax.lax.while_loop(_cond, _body, (0, lo, hi, cnt, chi))
    INF = 3.4028235e38
    u1 = jnp.min(jnp.where(s_sel >= lo, s_sel, INF), axis=1, keepdims=True)
    u2 = jnp.min(jnp.where(s_sel > u1, s_sel, INF), axis=1, keepdims=True)
    u3 = jnp.min(jnp.where(s_sel > u2, s_sel, INF), axis=1, keepdims=True)
    t = jnp.where(cnt == kf, lo, jnp.where(cnt == kf + 1.0, u2, u3))
    t = jnp.where(t >= INF, lo, t)
    z = jnp.where(s_sel >= t, s_sel * (qnorm * SCALE), NEG)
    null_s = _bdot(q, nk_ref[...].T) * SCALE
    m = jnp.maximum(jnp.max(z, axis=1, keepdims=True), null_s)
    w = jnp.exp(z - m)
    wn = jnp.exp(null_s - m)
    denom_m = jnp.sum(w, axis=1, keepdims=True) + wn
    mem = (_bdot(w, mv_ref[...]) + wn * nv_ref[...]) / denom_m

    g = jax.nn.sigmoid(gate_ref[0, 0, 0])
    part = loc * g + mem * (1.0 - g)
    contrib = _bdot(part, wout_ref[0])

    @pl.when(h == 0)
    def _init():
        out_ref[...] = contrib + bout_ref[...]

    @pl.when(h != 0)
    def _acc():
        out_ref[...] += contrib


@jax.jit
def kernel(x, mem_kv, W_q, W_kv, W_out, b_out, null_k, null_v, gate):
    x2 = x[0]
    mem_k = mem_kv[0, :, 0, :]
    mem_v = mem_kv[0, :, 1, :]
    wq_r = W_q.reshape(DIM, HEADS, DIM_HEAD).transpose(1, 0, 2)
    wout_r = W_out.reshape(HEADS, DIM_HEAD, DIM)

    kv = pl.pallas_call(
        _kv_kernel,
        grid=(1,),
        in_specs=[
            pl.BlockSpec((N, DIM), lambda i: (0, 0)),
            pl.BlockSpec((DIM, 2 * DIM_HEAD), lambda i: (0, 0)),
        ],
        out_specs=pl.BlockSpec((N, 2 * DIM_HEAD), lambda i: (0, 0)),
        out_shape=jax.ShapeDtypeStruct((N, 2 * DIM_HEAD), jnp.float32),
    )(x2, W_kv)

    out = pl.pallas_call(
        _attn_kernel,
        grid=(NI, HEADS),
        in_specs=[
            pl.BlockSpec((BLK, DIM), lambda i, h: (i, 0)),
            pl.BlockSpec((1, DIM, DIM_HEAD), lambda i, h: (h, 0, 0)),
            pl.BlockSpec((N, 2 * DIM_HEAD), lambda i, h: (0, 0)),
            pl.BlockSpec((M_MEM, DIM_HEAD), lambda i, h: (0, 0)),
            pl.BlockSpec((M_MEM, DIM_HEAD), lambda i, h: (0, 0)),
            pl.BlockSpec((1, DIM_HEAD), lambda i, h: (0, 0)),
            pl.BlockSpec((1, DIM_HEAD), lambda i, h: (0, 0)),
            pl.BlockSpec((1, 1, 1), lambda i, h: (h, 0, 0)),
            pl.BlockSpec((1, DIM_HEAD, DIM), lambda i, h: (h, 0, 0)),
            pl.BlockSpec((1, DIM), lambda i, h: (0, 0)),
        ],
        out_specs=pl.BlockSpec((BLK, DIM), lambda i, h: (i, 0)),
        out_shape=jax.ShapeDtypeStruct((N, DIM), jnp.float32),
        compiler_params=pltpu.CompilerParams(
            dimension_semantics=("parallel", "arbitrary"),
        ),
    )(x2, wq_r, kv, mem_k, mem_v,
      null_k.reshape(1, DIM_HEAD), null_v.reshape(1, DIM_HEAD),
      gate, wout_r, b_out.reshape(1, DIM))

    return out[None]
